# baseline (speedup 1.0000x reference)
import functools

import jax
import jax.numpy as jnp
from jax import lax
from jax.experimental import pallas as pl
from jax.experimental.pallas import tpu as pltpu

D_MODEL = 2048
N_EXP = 16
N_TOK = 16384
BLK = 1024
GRID = N_TOK // BLK


def _router_body(x_ref, wt_ref, w_ref, i_ref, aux_ref, facc, pacc):
    b = pl.program_id(0)
    xb = x_ref[...]
    lg = jnp.dot(xb, wt_ref[...], preferred_element_type=jnp.float32)

    iota = lax.broadcasted_iota(jnp.int32, (BLK, N_EXP), 1)
    m1 = jnp.max(lg, axis=1, keepdims=True)
    i1 = jnp.min(jnp.where(lg == m1, iota, N_EXP), axis=1, keepdims=True)
    masked = jnp.where(iota == i1, -jnp.inf, lg)
    m2 = jnp.max(masked, axis=1, keepdims=True)
    i2 = jnp.min(jnp.where(masked == m2, iota, N_EXP), axis=1, keepdims=True)

    t = jnp.exp(m2 - m1)
    w1 = 1.0 / (1.0 + t)
    w2 = t / (1.0 + t)
    w_ref[...] = jnp.concatenate([w1, w2], axis=1)
    i_ref[...] = jnp.concatenate([i1, i2], axis=1)

    e = jnp.exp(lg - m1)
    p = e / jnp.sum(e, axis=1, keepdims=True)

    @pl.when(b == 0)
    def _init():
        facc[...] = jnp.zeros_like(facc)
        pacc[...] = jnp.zeros_like(pacc)

    sel = (iota == i1).astype(jnp.float32) + (iota == i2).astype(jnp.float32)
    facc[...] += jnp.sum(sel, axis=0, keepdims=True)
    pacc[...] += jnp.sum(p, axis=0, keepdims=True)

    @pl.when(b == GRID - 1)
    def _fin():
        scale = N_EXP / (float(N_TOK) * float(N_TOK))
        aux_ref[0, 0] = scale * jnp.sum(facc[...] * pacc[...])


@jax.jit
def kernel(x, W):
    x_flat = x.reshape(N_TOK, D_MODEL)
    wt = W.T
    w_out, i_out, aux = pl.pallas_call(
        _router_body,
        grid=(GRID,),
        in_specs=[
            pl.BlockSpec((BLK, D_MODEL), lambda b: (b, 0)),
            pl.BlockSpec((D_MODEL, N_EXP), lambda b: (0, 0)),
        ],
        out_specs=[
            pl.BlockSpec((BLK, 2), lambda b: (b, 0)),
            pl.BlockSpec((BLK, 2), lambda b: (b, 0)),
            pl.BlockSpec(memory_space=pltpu.SMEM),
        ],
        out_shape=[
            jax.ShapeDtypeStruct((N_TOK, 2), jnp.float32),
            jax.ShapeDtypeStruct((N_TOK, 2), jnp.int32),
            jax.ShapeDtypeStruct((1, 1), jnp.float32),
        ],
        scratch_shapes=[
            pltpu.VMEM((1, N_EXP), jnp.float32),
            pltpu.VMEM((1, N_EXP), jnp.float32),
        ],
        compiler_params=pltpu.CompilerParams(
            dimension_semantics=("arbitrary",),
        ),
    )(x_flat, wt)
    return w_out, i_out, aux[0, 0]
